# baseline (speedup 1.0000x reference)
import functools

import jax
import jax.numpy as jnp
from jax import lax
from jax.experimental import pallas as pl
from jax.experimental.pallas import tpu as pltpu
from jax.experimental.pallas import tpu_sc as plsc

_N = 10000
_NP = 10240
_D = 128
_E = 320000
_NC = 2
_NS = 16
_NW = _NC * _NS
_CH = 125
_NCHUNK = 80
_KC = 8
_NG = _NCHUNK // _KC
_RPT = _NP // _NS

_B = 1024
_G = _NP // _B

_sc_mesh = plsc.VectorSubcoreMesh(core_axis_name="c", subcore_axis_name="s")


@functools.partial(
    pl.kernel,
    out_type=jax.ShapeDtypeStruct((_NC, _NP, 16), jnp.float32),
    mesh=_sc_mesh,
    compiler_params=pltpu.CompilerParams(use_tc_tiling_on_sc=False),
    scratch_types=[
        pltpu.VMEM((_NCHUNK, _CH), jnp.int32),
        pltpu.VMEM((_CH, 16), jnp.float32),
        pltpu.VMEM_SHARED((_NP, 16), jnp.float32),
    ],
)
def _deg_kernel(dst_hbm, ones_hbm, z16_hbm, cnt_hbm, dst_v, ones_v, acc):
    c = lax.axis_index("c")
    s = lax.axis_index("s")
    wid = c * _NS + s
    pltpu.sync_copy(dst_hbm.at[wid], dst_v)
    pltpu.sync_copy(ones_hbm, ones_v)
    r0 = s * _RPT
    pltpu.sync_copy(z16_hbm.at[pl.ds(r0, _RPT)], acc.at[pl.ds(r0, _RPT)])
    plsc.subcore_barrier()

    def body(j, carry):
        pltpu.sync_copy(ones_v, acc.at[dst_v.at[j]], add=True)
        return carry

    lax.fori_loop(0, _NCHUNK, body, 0)
    plsc.subcore_barrier()
    pltpu.sync_copy(acc.at[pl.ds(r0, _RPT)], cnt_hbm.at[c, pl.ds(r0, _RPT)])


@functools.partial(
    pl.kernel,
    out_type=jax.ShapeDtypeStruct((_NC, _NP, _D), jnp.float32),
    mesh=_sc_mesh,
    scratch_types=[
        pltpu.VMEM((2, _KC, _CH), jnp.int32),
        pltpu.VMEM((2, _KC, _CH), jnp.int32),
        pltpu.VMEM((2, _CH, _D), jnp.float32),
        pltpu.VMEM_SHARED((_NP, _D), jnp.float32),
        pltpu.SemaphoreType.DMA,
        pltpu.SemaphoreType.DMA,
        pltpu.SemaphoreType.DMA,
    ],
)
def _edge_kernel(ys_hbm, src_hbm, dst_hbm, z128_hbm, acc_hbm,
                 sidx, didx, rows_v, acc, gsem0, gsem1, isem):
    c = lax.axis_index("c")
    s = lax.axis_index("s")
    wid = c * _NS + s
    r0 = s * _RPT

    @pl.when(c == 0)
    def _():
        pltpu.sync_copy(ys_hbm.at[pl.ds(r0, _RPT)], acc.at[pl.ds(r0, _RPT)])

    @pl.when(c == 1)
    def _():
        pltpu.sync_copy(z128_hbm.at[pl.ds(r0, _RPT)], acc.at[pl.ds(r0, _RPT)])

    plsc.subcore_barrier()

    gsems = (gsem0, gsem1)
    pltpu.sync_copy(src_hbm.at[wid, pl.ds(0, _KC)], sidx.at[0])
    pltpu.sync_copy(dst_hbm.at[wid, pl.ds(0, _KC)], didx.at[0])
    pltpu.async_copy(ys_hbm.at[sidx.at[0, 0]], rows_v.at[0], gsem0)

    def gbody(g, carry):
        gb = lax.rem(g, 2)
        gn = lax.rem(g + 1, 2)
        gsrc = lax.rem(g + 1, _NG)
        isrc = pltpu.async_copy(src_hbm.at[wid, pl.ds(gsrc * _KC, _KC)],
                                sidx.at[gn], isem)
        idst = pltpu.async_copy(dst_hbm.at[wid, pl.ds(gsrc * _KC, _KC)],
                                didx.at[gn], isem)
        pending = None
        for k in range(_KC):
            cb = k % 2
            nb = (k + 1) % 2
            if k < _KC - 1:
                nxt = pltpu.async_copy(ys_hbm.at[sidx.at[gb, k + 1]],
                                       rows_v.at[nb], gsems[nb])
            else:
                isrc.wait()
                idst.wait()
                nxt = pltpu.async_copy(ys_hbm.at[sidx.at[gn, 0]],
                                       rows_v.at[nb], gsems[nb])
            if pending is None:
                pltpu.make_async_copy(ys_hbm.at[sidx.at[gb, k]],
                                      rows_v.at[cb], gsems[cb]).wait()
            else:
                pending.wait()
            pending = nxt
            pltpu.sync_copy(rows_v.at[cb], acc.at[didx.at[gb, k]], add=True)
        return carry

    lax.fori_loop(0, _NG, gbody, 0)
    pltpu.make_async_copy(ys_hbm.at[sidx.at[0, 0]], rows_v.at[0],
                          gsem0).wait()
    plsc.subcore_barrier()
    pltpu.sync_copy(acc.at[pl.ds(r0, _RPT)], acc_hbm.at[c, pl.ds(r0, _RPT)])


def _dinv_block(cnt_blk):
    deg = cnt_blk[0, :, 0:1] + cnt_blk[1, :, 0:1] + 1.0
    return lax.rsqrt(deg)


def _k0_body(x_ref, wl_ref, bl_ref, h_ref):
    h_ref[...] = jnp.maximum(
        jnp.dot(x_ref[...], wl_ref[...], preferred_element_type=jnp.float32)
        + bl_ref[...], 0.0)


def _k1_body(h_ref, w1_ref, cnt_ref, ys_ref):
    dinv = _dinv_block(cnt_ref[...])
    ys_ref[...] = jnp.dot(h_ref[...], w1_ref[...],
                          preferred_element_type=jnp.float32) * dinv


def _kb_body(acc_ref, cnt_ref, b_ref, w_ref, ys_ref):
    dinv = _dinv_block(cnt_ref[...])
    h = jnp.maximum(
        (acc_ref[0, :, :] + acc_ref[1, :, :]) * dinv + b_ref[...], 0.0)
    ys_ref[...] = jnp.dot(h, w_ref[...],
                          preferred_element_type=jnp.float32) * dinv


def _kc_body(acc_ref, cnt_ref, b_ref, out_ref):
    dinv = _dinv_block(cnt_ref[...])
    out_ref[...] = jnp.maximum(
        (acc_ref[0, :, :] + acc_ref[1, :, :]) * dinv + b_ref[...], 0.0)


_row_spec = pl.BlockSpec((_B, _D), lambda i: (i, 0))
_mat_spec = pl.BlockSpec((_D, _D), lambda i: (0, 0))
_vec_spec = pl.BlockSpec((_D,), lambda i: (0,))
_cnt_spec = pl.BlockSpec((_NC, _B, 16), lambda i: (0, i, 0))
_acc_spec = pl.BlockSpec((_NC, _B, _D), lambda i: (0, i, 0))
_out_sds = jax.ShapeDtypeStruct((_NP, _D), jnp.float32)


def _k0(x, wl, bl):
    return pl.pallas_call(
        _k0_body, grid=(_G,),
        in_specs=[_row_spec, _mat_spec, _vec_spec],
        out_specs=_row_spec, out_shape=_out_sds,
    )(x, wl, bl)


def _k1(h, w1, cnt):
    return pl.pallas_call(
        _k1_body, grid=(_G,),
        in_specs=[_row_spec, _mat_spec, _cnt_spec],
        out_specs=_row_spec, out_shape=_out_sds,
    )(h, w1, cnt)


def _kb(acc, cnt, b, w):
    return pl.pallas_call(
        _kb_body, grid=(_G,),
        in_specs=[_acc_spec, _cnt_spec, _vec_spec, _mat_spec],
        out_specs=_row_spec, out_shape=_out_sds,
    )(acc, cnt, b, w)


_BF = 1000


def _kc(acc, cnt, b):
    return pl.pallas_call(
        _kc_body, grid=(_N // _BF,),
        in_specs=[
            pl.BlockSpec((_NC, _BF, _D), lambda i: (0, i, 0)),
            pl.BlockSpec((_NC, _BF, 16), lambda i: (0, i, 0)),
            pl.BlockSpec((_D,), lambda i: (0,)),
        ],
        out_specs=pl.BlockSpec((_BF, _D), lambda i: (i, 0)),
        out_shape=jax.ShapeDtypeStruct((_N, _D), jnp.float32),
    )(acc, cnt, b)


def kernel(features, edge_index, lin_w, lin_b, gcn_w, gcn_b):
    src = edge_index[0].reshape(_NW, _NCHUNK, _CH)
    dst = edge_index[1].reshape(_NW, _NCHUNK, _CH)
    xp = jnp.pad(features, ((0, _NP - _N), (0, 0)))
    ones16 = jnp.ones((_CH, 16), jnp.float32)
    z16 = jnp.zeros((_NP, 16), jnp.float32)
    z128 = jnp.zeros((_NP, _D), jnp.float32)

    h0 = _k0(xp, lin_w, lin_b)
    cnt = _deg_kernel(dst, ones16, z16)
    ys = _k1(h0, gcn_w[0], cnt)
    acc = _edge_kernel(ys, src, dst, z128)
    ys = _kb(acc, cnt, gcn_b[0], gcn_w[1])
    acc = _edge_kernel(ys, src, dst, z128)
    return _kc(acc, cnt, gcn_b[1])


# speedup vs baseline: 1.0177x; 1.0177x over previous
import functools

import jax
import jax.numpy as jnp
from jax import lax
from jax.experimental import pallas as pl
from jax.experimental.pallas import tpu as pltpu
from jax.experimental.pallas import tpu_sc as plsc

_N = 10000
_NP = 10240
_D = 128
_E = 320000
_NC = 2
_NS = 16
_NW = _NC * _NS
_CH = 125
_NCHUNK = 80
_KC = 8
_NG = _NCHUNK // _KC
_RPT = _NP // _NS

_B = 2048
_G = _NP // _B

_sc_mesh = plsc.VectorSubcoreMesh(core_axis_name="c", subcore_axis_name="s")


@functools.partial(
    pl.kernel,
    out_type=jax.ShapeDtypeStruct((_NC, _NP, 16), jnp.float32),
    mesh=_sc_mesh,
    compiler_params=pltpu.CompilerParams(use_tc_tiling_on_sc=False),
    scratch_types=[
        pltpu.VMEM((_NCHUNK, _CH), jnp.int32),
        pltpu.VMEM((_CH, 16), jnp.float32),
        pltpu.VMEM_SHARED((_NP, 16), jnp.float32),
    ],
)
def _deg_kernel(dst_hbm, ones_hbm, z16_hbm, cnt_hbm, dst_v, ones_v, acc):
    c = lax.axis_index("c")
    s = lax.axis_index("s")
    wid = c * _NS + s
    pltpu.sync_copy(dst_hbm.at[wid], dst_v)
    pltpu.sync_copy(ones_hbm, ones_v)
    r0 = s * _RPT
    pltpu.sync_copy(z16_hbm.at[pl.ds(r0, _RPT)], acc.at[pl.ds(r0, _RPT)])
    plsc.subcore_barrier()

    def body(j, carry):
        pltpu.sync_copy(ones_v, acc.at[dst_v.at[j]], add=True)
        return carry

    lax.fori_loop(0, _NCHUNK, body, 0)
    plsc.subcore_barrier()
    pltpu.sync_copy(acc.at[pl.ds(r0, _RPT)], cnt_hbm.at[c, pl.ds(r0, _RPT)])


@functools.partial(
    pl.kernel,
    out_type=jax.ShapeDtypeStruct((_NC, _NP, _D), jnp.float32),
    mesh=_sc_mesh,
    scratch_types=[
        pltpu.VMEM((2, _KC, _CH), jnp.int32),
        pltpu.VMEM((2, _KC, _CH), jnp.int32),
        pltpu.VMEM((2, _CH, _D), jnp.float32),
        pltpu.VMEM_SHARED((_NP, _D), jnp.float32),
        pltpu.SemaphoreType.DMA,
        pltpu.SemaphoreType.DMA,
        pltpu.SemaphoreType.DMA,
    ],
)
def _edge_kernel(ys_hbm, src_hbm, dst_hbm, z128_hbm, acc_hbm,
                 sidx, didx, rows_v, acc, gsem0, gsem1, isem):
    c = lax.axis_index("c")
    s = lax.axis_index("s")
    wid = c * _NS + s
    r0 = s * _RPT

    @pl.when(c == 0)
    def _():
        pltpu.sync_copy(ys_hbm.at[pl.ds(r0, _RPT)], acc.at[pl.ds(r0, _RPT)])

    @pl.when(c == 1)
    def _():
        pltpu.sync_copy(z128_hbm.at[pl.ds(r0, _RPT)], acc.at[pl.ds(r0, _RPT)])

    plsc.subcore_barrier()

    gsems = (gsem0, gsem1)
    pltpu.sync_copy(src_hbm.at[wid, pl.ds(0, _KC)], sidx.at[0])
    pltpu.sync_copy(dst_hbm.at[wid, pl.ds(0, _KC)], didx.at[0])
    pltpu.async_copy(ys_hbm.at[sidx.at[0, 0]], rows_v.at[0], gsem0)

    def gbody(g, carry):
        gb = lax.rem(g, 2)
        gn = lax.rem(g + 1, 2)
        gsrc = lax.rem(g + 1, _NG)
        isrc = pltpu.async_copy(src_hbm.at[wid, pl.ds(gsrc * _KC, _KC)],
                                sidx.at[gn], isem)
        idst = pltpu.async_copy(dst_hbm.at[wid, pl.ds(gsrc * _KC, _KC)],
                                didx.at[gn], isem)
        pending = None
        for k in range(_KC):
            cb = k % 2
            nb = (k + 1) % 2
            if k < _KC - 1:
                nxt = pltpu.async_copy(ys_hbm.at[sidx.at[gb, k + 1]],
                                       rows_v.at[nb], gsems[nb])
            else:
                isrc.wait()
                idst.wait()
                nxt = pltpu.async_copy(ys_hbm.at[sidx.at[gn, 0]],
                                       rows_v.at[nb], gsems[nb])
            if pending is None:
                pltpu.make_async_copy(ys_hbm.at[sidx.at[gb, k]],
                                      rows_v.at[cb], gsems[cb]).wait()
            else:
                pending.wait()
            pending = nxt
            pltpu.sync_copy(rows_v.at[cb], acc.at[didx.at[gb, k]], add=True)
        return carry

    lax.fori_loop(0, _NG, gbody, 0)
    pltpu.make_async_copy(ys_hbm.at[sidx.at[0, 0]], rows_v.at[0],
                          gsem0).wait()
    plsc.subcore_barrier()
    pltpu.sync_copy(acc.at[pl.ds(r0, _RPT)], acc_hbm.at[c, pl.ds(r0, _RPT)])


def _dinv_block(cnt_blk):
    deg = cnt_blk[0, :, 0:1] + cnt_blk[1, :, 0:1] + 1.0
    return lax.rsqrt(deg)


def _ka_body(x_ref, wl_ref, bl_ref, w1_ref, cnt_ref, ys_ref):
    h = jnp.maximum(
        jnp.dot(x_ref[...], wl_ref[...], preferred_element_type=jnp.float32)
        + bl_ref[...], 0.0)
    dinv = _dinv_block(cnt_ref[...])
    ys_ref[...] = jnp.dot(h, w1_ref[...],
                          preferred_element_type=jnp.float32) * dinv


def _kb_body(acc_ref, cnt_ref, b_ref, w_ref, ys_ref):
    dinv = _dinv_block(cnt_ref[...])
    h = jnp.maximum(
        (acc_ref[0, :, :] + acc_ref[1, :, :]) * dinv + b_ref[...], 0.0)
    ys_ref[...] = jnp.dot(h, w_ref[...],
                          preferred_element_type=jnp.float32) * dinv


def _kc_body(acc_ref, cnt_ref, b_ref, out_ref):
    dinv = _dinv_block(cnt_ref[...])
    out_ref[...] = jnp.maximum(
        (acc_ref[0, :, :] + acc_ref[1, :, :]) * dinv + b_ref[...], 0.0)


_row_spec = pl.BlockSpec((_B, _D), lambda i: (i, 0))
_mat_spec = pl.BlockSpec((_D, _D), lambda i: (0, 0))
_vec_spec = pl.BlockSpec((_D,), lambda i: (0,))
_cnt_spec = pl.BlockSpec((_NC, _B, 16), lambda i: (0, i, 0))
_acc_spec = pl.BlockSpec((_NC, _B, _D), lambda i: (0, i, 0))
_out_sds = jax.ShapeDtypeStruct((_NP, _D), jnp.float32)


def _ka(x, wl, bl, w1, cnt):
    return pl.pallas_call(
        _ka_body, grid=(_G,),
        in_specs=[_row_spec, _mat_spec, _vec_spec, _mat_spec, _cnt_spec],
        out_specs=_row_spec, out_shape=_out_sds,
    )(x, wl, bl, w1, cnt)


def _kb(acc, cnt, b, w):
    return pl.pallas_call(
        _kb_body, grid=(_G,),
        in_specs=[_acc_spec, _cnt_spec, _vec_spec, _mat_spec],
        out_specs=_row_spec, out_shape=_out_sds,
    )(acc, cnt, b, w)


_BF = 1000


def _kc(acc, cnt, b):
    return pl.pallas_call(
        _kc_body, grid=(_N // _BF,),
        in_specs=[
            pl.BlockSpec((_NC, _BF, _D), lambda i: (0, i, 0)),
            pl.BlockSpec((_NC, _BF, 16), lambda i: (0, i, 0)),
            pl.BlockSpec((_D,), lambda i: (0,)),
        ],
        out_specs=pl.BlockSpec((_BF, _D), lambda i: (i, 0)),
        out_shape=jax.ShapeDtypeStruct((_N, _D), jnp.float32),
    )(acc, cnt, b)


def kernel(features, edge_index, lin_w, lin_b, gcn_w, gcn_b):
    src = edge_index[0].reshape(_NW, _NCHUNK, _CH)
    dst = edge_index[1].reshape(_NW, _NCHUNK, _CH)
    xp = jnp.pad(features, ((0, _NP - _N), (0, 0)))
    ones16 = jnp.ones((_CH, 16), jnp.float32)
    z16 = jnp.zeros((_NP, 16), jnp.float32)
    z128 = jnp.zeros((_NP, _D), jnp.float32)

    cnt = _deg_kernel(dst, ones16, z16)
    ys = _ka(xp, lin_w, lin_b, gcn_w[0], cnt)
    acc = _edge_kernel(ys, src, dst, z128)
    ys = _kb(acc, cnt, gcn_b[0], gcn_w[1])
    acc = _edge_kernel(ys, src, dst, z128)
    return _kc(acc, cnt, gcn_b[1])


# speedup vs baseline: 1.0253x; 1.0074x over previous
import functools

import jax
import jax.numpy as jnp
from jax import lax
from jax.experimental import pallas as pl
from jax.experimental.pallas import tpu as pltpu
from jax.experimental.pallas import tpu_sc as plsc

_N = 10000
_NP = 10240
_D = 128
_E = 320000
_NC = 2
_NS = 16
_NW = _NC * _NS
_CH = 125
_NCHUNK = 80
_KC = 8
_NG = _NCHUNK // _KC
_RPT = _NP // _NS

_B = 5120
_G = _NP // _B

_sc_mesh = plsc.VectorSubcoreMesh(core_axis_name="c", subcore_axis_name="s")


@functools.partial(
    pl.kernel,
    out_type=jax.ShapeDtypeStruct((_NC, _NP, 16), jnp.float32),
    mesh=_sc_mesh,
    compiler_params=pltpu.CompilerParams(use_tc_tiling_on_sc=False),
    scratch_types=[
        pltpu.VMEM((_NCHUNK, _CH), jnp.int32),
        pltpu.VMEM((_CH, 16), jnp.float32),
        pltpu.VMEM_SHARED((_NP, 16), jnp.float32),
    ],
)
def _deg_kernel(dst_hbm, ones_hbm, z16_hbm, cnt_hbm, dst_v, ones_v, acc):
    c = lax.axis_index("c")
    s = lax.axis_index("s")
    wid = c * _NS + s
    pltpu.sync_copy(dst_hbm.at[wid], dst_v)
    pltpu.sync_copy(ones_hbm, ones_v)
    r0 = s * _RPT
    pltpu.sync_copy(z16_hbm.at[pl.ds(r0, _RPT)], acc.at[pl.ds(r0, _RPT)])
    plsc.subcore_barrier()

    def body(j, carry):
        pltpu.sync_copy(ones_v, acc.at[dst_v.at[j]], add=True)
        return carry

    lax.fori_loop(0, _NCHUNK, body, 0)
    plsc.subcore_barrier()
    pltpu.sync_copy(acc.at[pl.ds(r0, _RPT)], cnt_hbm.at[c, pl.ds(r0, _RPT)])


@functools.partial(
    pl.kernel,
    out_type=jax.ShapeDtypeStruct((_NC, _NP, _D), jnp.float32),
    mesh=_sc_mesh,
    scratch_types=[
        pltpu.VMEM((2, _KC, _CH), jnp.int32),
        pltpu.VMEM((2, _KC, _CH), jnp.int32),
        pltpu.VMEM((2, _CH, _D), jnp.float32),
        pltpu.VMEM_SHARED((_NP, _D), jnp.float32),
        pltpu.SemaphoreType.DMA,
        pltpu.SemaphoreType.DMA,
        pltpu.SemaphoreType.DMA,
    ],
)
def _edge_kernel(ys_hbm, src_hbm, dst_hbm, z128_hbm, acc_hbm,
                 sidx, didx, rows_v, acc, gsem0, gsem1, isem):
    c = lax.axis_index("c")
    s = lax.axis_index("s")
    wid = c * _NS + s
    r0 = s * _RPT

    @pl.when(c == 0)
    def _():
        pltpu.sync_copy(ys_hbm.at[pl.ds(r0, _RPT)], acc.at[pl.ds(r0, _RPT)])

    @pl.when(c == 1)
    def _():
        pltpu.sync_copy(z128_hbm.at[pl.ds(r0, _RPT)], acc.at[pl.ds(r0, _RPT)])

    plsc.subcore_barrier()

    gsems = (gsem0, gsem1)
    pltpu.sync_copy(src_hbm.at[wid, pl.ds(0, _KC)], sidx.at[0])
    pltpu.sync_copy(dst_hbm.at[wid, pl.ds(0, _KC)], didx.at[0])
    pltpu.async_copy(ys_hbm.at[sidx.at[0, 0]], rows_v.at[0], gsem0)

    def gbody(g, carry):
        gb = lax.rem(g, 2)
        gn = lax.rem(g + 1, 2)
        gsrc = lax.rem(g + 1, _NG)
        isrc = pltpu.async_copy(src_hbm.at[wid, pl.ds(gsrc * _KC, _KC)],
                                sidx.at[gn], isem)
        idst = pltpu.async_copy(dst_hbm.at[wid, pl.ds(gsrc * _KC, _KC)],
                                didx.at[gn], isem)
        pending = None
        for k in range(_KC):
            cb = k % 2
            nb = (k + 1) % 2
            if k < _KC - 1:
                nxt = pltpu.async_copy(ys_hbm.at[sidx.at[gb, k + 1]],
                                       rows_v.at[nb], gsems[nb])
            else:
                isrc.wait()
                idst.wait()
                nxt = pltpu.async_copy(ys_hbm.at[sidx.at[gn, 0]],
                                       rows_v.at[nb], gsems[nb])
            if pending is None:
                pltpu.make_async_copy(ys_hbm.at[sidx.at[gb, k]],
                                      rows_v.at[cb], gsems[cb]).wait()
            else:
                pending.wait()
            pending = nxt
            pltpu.sync_copy(rows_v.at[cb], acc.at[didx.at[gb, k]], add=True)
        return carry

    lax.fori_loop(0, _NG, gbody, 0)
    pltpu.make_async_copy(ys_hbm.at[sidx.at[0, 0]], rows_v.at[0],
                          gsem0).wait()
    plsc.subcore_barrier()
    pltpu.sync_copy(acc.at[pl.ds(r0, _RPT)], acc_hbm.at[c, pl.ds(r0, _RPT)])


def _dinv_block(cnt_blk):
    deg = cnt_blk[0, :, 0:1] + cnt_blk[1, :, 0:1] + 1.0
    return lax.rsqrt(deg)


def _ka_body(x_ref, wl_ref, bl_ref, w1_ref, cnt_ref, ys_ref):
    h = jnp.maximum(
        jnp.dot(x_ref[...], wl_ref[...], preferred_element_type=jnp.float32)
        + bl_ref[...], 0.0)
    dinv = _dinv_block(cnt_ref[...])
    ys_ref[...] = jnp.dot(h, w1_ref[...],
                          preferred_element_type=jnp.float32) * dinv


def _kb_body(acc_ref, cnt_ref, b_ref, w_ref, ys_ref):
    dinv = _dinv_block(cnt_ref[...])
    h = jnp.maximum(
        (acc_ref[0, :, :] + acc_ref[1, :, :]) * dinv + b_ref[...], 0.0)
    ys_ref[...] = jnp.dot(h, w_ref[...],
                          preferred_element_type=jnp.float32) * dinv


def _kc_body(acc_ref, cnt_ref, b_ref, out_ref):
    dinv = _dinv_block(cnt_ref[...])
    out_ref[...] = jnp.maximum(
        (acc_ref[0, :, :] + acc_ref[1, :, :]) * dinv + b_ref[...], 0.0)


_row_spec = pl.BlockSpec((_B, _D), lambda i: (i, 0))
_mat_spec = pl.BlockSpec((_D, _D), lambda i: (0, 0))
_vec_spec = pl.BlockSpec((_D,), lambda i: (0,))
_cnt_spec = pl.BlockSpec((_NC, _B, 16), lambda i: (0, i, 0))
_acc_spec = pl.BlockSpec((_NC, _B, _D), lambda i: (0, i, 0))
_out_sds = jax.ShapeDtypeStruct((_NP, _D), jnp.float32)


def _ka(x, wl, bl, w1, cnt):
    return pl.pallas_call(
        _ka_body, grid=(_G,),
        in_specs=[_row_spec, _mat_spec, _vec_spec, _mat_spec, _cnt_spec],
        out_specs=_row_spec, out_shape=_out_sds,
    )(x, wl, bl, w1, cnt)


def _kb(acc, cnt, b, w):
    return pl.pallas_call(
        _kb_body, grid=(_G,),
        in_specs=[_acc_spec, _cnt_spec, _vec_spec, _mat_spec],
        out_specs=_row_spec, out_shape=_out_sds,
    )(acc, cnt, b, w)


_BF = 1000


def _kc(acc, cnt, b):
    return pl.pallas_call(
        _kc_body, grid=(_N // _BF,),
        in_specs=[
            pl.BlockSpec((_NC, _BF, _D), lambda i: (0, i, 0)),
            pl.BlockSpec((_NC, _BF, 16), lambda i: (0, i, 0)),
            pl.BlockSpec((_D,), lambda i: (0,)),
        ],
        out_specs=pl.BlockSpec((_BF, _D), lambda i: (i, 0)),
        out_shape=jax.ShapeDtypeStruct((_N, _D), jnp.float32),
    )(acc, cnt, b)


def kernel(features, edge_index, lin_w, lin_b, gcn_w, gcn_b):
    src = edge_index[0].reshape(_NW, _NCHUNK, _CH)
    dst = edge_index[1].reshape(_NW, _NCHUNK, _CH)
    xp = jnp.pad(features, ((0, _NP - _N), (0, 0)))
    ones16 = jnp.ones((_CH, 16), jnp.float32)
    z16 = jnp.zeros((_NP, 16), jnp.float32)
    z128 = jnp.zeros((_NP, _D), jnp.float32)

    cnt = _deg_kernel(dst, ones16, z16)
    ys = _ka(xp, lin_w, lin_b, gcn_w[0], cnt)
    acc = _edge_kernel(ys, src, dst, z128)
    ys = _kb(acc, cnt, gcn_b[0], gcn_w[1])
    acc = _edge_kernel(ys, src, dst, z128)
    return _kc(acc, cnt, gcn_b[1])


# speedup vs baseline: 1.0348x; 1.0093x over previous
import functools

import jax
import jax.numpy as jnp
from jax import lax
from jax.experimental import pallas as pl
from jax.experimental.pallas import tpu as pltpu
from jax.experimental.pallas import tpu_sc as plsc

_N = 10000
_NP = 10240
_D = 128
_E = 320000
_NC = 2
_NS = 16
_NW = _NC * _NS
_CH = 125
_NCHUNK = 80
_KC = 8
_NG = _NCHUNK // _KC
_RPT = _NP // _NS

_B = 5120
_G = _NP // _B

_sc_mesh = plsc.VectorSubcoreMesh(core_axis_name="c", subcore_axis_name="s")


@functools.partial(
    pl.kernel,
    out_type=jax.ShapeDtypeStruct((_NC, _NP, 16), jnp.float32),
    mesh=_sc_mesh,
    compiler_params=pltpu.CompilerParams(use_tc_tiling_on_sc=False),
    scratch_types=[
        pltpu.VMEM((_NCHUNK, _CH), jnp.int32),
        pltpu.VMEM((_CH, 16), jnp.float32),
        pltpu.VMEM_SHARED((_NP, 16), jnp.float32),
    ],
)
def _deg_kernel(dst_hbm, ones_hbm, z16_hbm, cnt_hbm, dst_v, ones_v, acc):
    c = lax.axis_index("c")
    s = lax.axis_index("s")
    wid = c * _NS + s
    pltpu.sync_copy(dst_hbm.at[wid], dst_v)
    pltpu.sync_copy(ones_hbm, ones_v)
    r0 = s * _RPT
    pltpu.sync_copy(z16_hbm.at[pl.ds(r0, _RPT)], acc.at[pl.ds(r0, _RPT)])
    plsc.subcore_barrier()

    def body(j, carry):
        pltpu.sync_copy(ones_v, acc.at[dst_v.at[j]], add=True)
        return carry

    lax.fori_loop(0, _NCHUNK, body, 0)
    plsc.subcore_barrier()
    pltpu.sync_copy(acc.at[pl.ds(r0, _RPT)], cnt_hbm.at[c, pl.ds(r0, _RPT)])


@functools.partial(
    pl.kernel,
    out_type=jax.ShapeDtypeStruct((_NC, _NP, _D), jnp.float32),
    mesh=_sc_mesh,
    scratch_types=[
        pltpu.VMEM((2, _KC, _CH), jnp.int32),
        pltpu.VMEM((2, _KC, _CH), jnp.int32),
        pltpu.VMEM((2, _CH, _D), jnp.float32),
        pltpu.VMEM_SHARED((_NP, _D), jnp.float32),
        pltpu.SemaphoreType.DMA,
        pltpu.SemaphoreType.DMA,
        pltpu.SemaphoreType.DMA,
    ],
)
def _edge_kernel(ys_hbm, src_hbm, dst_hbm, z128_hbm, acc_hbm,
                 sidx, didx, rows_v, acc, gsem0, gsem1, isem):
    c = lax.axis_index("c")
    s = lax.axis_index("s")
    wid = c * _NS + s
    r0 = s * _RPT

    @pl.when(c == 0)
    def _():
        pltpu.sync_copy(ys_hbm.at[pl.ds(r0, _RPT)], acc.at[pl.ds(r0, _RPT)])

    @pl.when(c == 1)
    def _():
        pltpu.sync_copy(z128_hbm.at[pl.ds(r0, _RPT)], acc.at[pl.ds(r0, _RPT)])

    plsc.subcore_barrier()

    gsems = (gsem0, gsem1)
    pltpu.sync_copy(src_hbm.at[wid, pl.ds(0, _KC)], sidx.at[0])
    pltpu.sync_copy(dst_hbm.at[wid, pl.ds(0, _KC)], didx.at[0])
    pltpu.async_copy(ys_hbm.at[sidx.at[0, 0]], rows_v.at[0], gsem0)

    def gbody(g, carry):
        gb = lax.rem(g, 2)
        gn = lax.rem(g + 1, 2)
        gsrc = lax.rem(g + 1, _NG)
        isrc = pltpu.async_copy(src_hbm.at[wid, pl.ds(gsrc * _KC, _KC)],
                                sidx.at[gn], isem)
        idst = pltpu.async_copy(dst_hbm.at[wid, pl.ds(gsrc * _KC, _KC)],
                                didx.at[gn], isem)
        pending = None
        for k in range(_KC):
            cb = k % 2
            nb = (k + 1) % 2
            if k < _KC - 1:
                nxt = pltpu.async_copy(ys_hbm.at[sidx.at[gb, k + 1]],
                                       rows_v.at[nb], gsems[nb])
            else:
                isrc.wait()
                idst.wait()
                nxt = pltpu.async_copy(ys_hbm.at[sidx.at[gn, 0]],
                                       rows_v.at[nb], gsems[nb])
            if pending is None:
                pltpu.make_async_copy(ys_hbm.at[sidx.at[gb, k]],
                                      rows_v.at[cb], gsems[cb]).wait()
            else:
                pending.wait()
            pending = nxt
            pltpu.sync_copy(rows_v.at[cb], acc.at[didx.at[gb, k]], add=True)
        return carry

    lax.fori_loop(0, _NG, gbody, 0)
    pltpu.make_async_copy(ys_hbm.at[sidx.at[0, 0]], rows_v.at[0],
                          gsem0).wait()
    plsc.subcore_barrier()
    pltpu.sync_copy(acc.at[pl.ds(r0, _RPT)], acc_hbm.at[c, pl.ds(r0, _RPT)])


def _dinv_block(cnt_blk):
    deg = cnt_blk[0, :, 0:1] + cnt_blk[1, :, 0:1] + 1.0
    return lax.rsqrt(deg)


def _ka_body(x_ref, wl_ref, bl_ref, w1_ref, cnt_ref, ys_ref):
    h = jnp.maximum(
        jnp.dot(x_ref[...], wl_ref[...], preferred_element_type=jnp.float32)
        + bl_ref[...], 0.0)
    dinv = _dinv_block(cnt_ref[...])
    ys_ref[...] = jnp.dot(h, w1_ref[...],
                          preferred_element_type=jnp.float32) * dinv


def _kb_body(acc_ref, cnt_ref, b_ref, w_ref, ys_ref):
    dinv = _dinv_block(cnt_ref[...])
    h = jnp.maximum(
        (acc_ref[0, :, :] + acc_ref[1, :, :]) * dinv + b_ref[...], 0.0)
    ys_ref[...] = jnp.dot(h, w_ref[...],
                          preferred_element_type=jnp.float32) * dinv


def _kc_body(acc_ref, cnt_ref, b_ref, out_ref):
    dinv = _dinv_block(cnt_ref[...])
    out_ref[...] = jnp.maximum(
        (acc_ref[0, :, :] + acc_ref[1, :, :]) * dinv + b_ref[...], 0.0)


_row_spec = pl.BlockSpec((_B, _D), lambda i: (i, 0))
_mat_spec = pl.BlockSpec((_D, _D), lambda i: (0, 0))
_vec_spec = pl.BlockSpec((_D,), lambda i: (0,))
_cnt_spec = pl.BlockSpec((_NC, _B, 16), lambda i: (0, i, 0))
_acc_spec = pl.BlockSpec((_NC, _B, _D), lambda i: (0, i, 0))
_out_sds = jax.ShapeDtypeStruct((_NP, _D), jnp.float32)


def _ka(x, wl, bl, w1, cnt):
    return pl.pallas_call(
        _ka_body, grid=(_G,),
        in_specs=[_row_spec, _mat_spec, _vec_spec, _mat_spec, _cnt_spec],
        out_specs=_row_spec, out_shape=_out_sds,
    )(x, wl, bl, w1, cnt)


def _kb(acc, cnt, b, w):
    return pl.pallas_call(
        _kb_body, grid=(_G,),
        in_specs=[_acc_spec, _cnt_spec, _vec_spec, _mat_spec],
        out_specs=_row_spec, out_shape=_out_sds,
    )(acc, cnt, b, w)


_BF = 5000


def _kc(acc, cnt, b):
    return pl.pallas_call(
        _kc_body, grid=(_N // _BF,),
        in_specs=[
            pl.BlockSpec((_NC, _BF, _D), lambda i: (0, i, 0)),
            pl.BlockSpec((_NC, _BF, 16), lambda i: (0, i, 0)),
            pl.BlockSpec((_D,), lambda i: (0,)),
        ],
        out_specs=pl.BlockSpec((_BF, _D), lambda i: (i, 0)),
        out_shape=jax.ShapeDtypeStruct((_N, _D), jnp.float32),
    )(acc, cnt, b)


def kernel(features, edge_index, lin_w, lin_b, gcn_w, gcn_b):
    src = edge_index[0].reshape(_NW, _NCHUNK, _CH)
    dst = edge_index[1].reshape(_NW, _NCHUNK, _CH)
    xp = jnp.pad(features, ((0, _NP - _N), (0, 0)))
    ones16 = jnp.ones((_CH, 16), jnp.float32)
    z16 = jnp.zeros((_NP, 16), jnp.float32)
    z128 = jnp.zeros((_NP, _D), jnp.float32)

    cnt = _deg_kernel(dst, ones16, z16)
    ys = _ka(xp, lin_w, lin_b, gcn_w[0], cnt)
    acc = _edge_kernel(ys, src, dst, z128)
    ys = _kb(acc, cnt, gcn_b[0], gcn_w[1])
    acc = _edge_kernel(ys, src, dst, z128)
    return _kc(acc, cnt, gcn_b[1])


# speedup vs baseline: 1.0457x; 1.0105x over previous
import functools

import jax
import jax.numpy as jnp
from jax import lax
from jax.experimental import pallas as pl
from jax.experimental.pallas import tpu as pltpu
from jax.experimental.pallas import tpu_sc as plsc

_N = 10000
_NP = 10240
_D = 128
_E = 320000
_NC = 2
_NS = 16
_NW = _NC * _NS
_CH = 125
_NCHUNK = 80
_KC = 8
_NG = _NCHUNK // _KC
_RPT = _NP // _NS

_B = 5120
_G = _NP // _B

_sc_mesh = plsc.VectorSubcoreMesh(core_axis_name="c", subcore_axis_name="s")


@functools.partial(
    pl.kernel,
    out_type=jax.ShapeDtypeStruct((_NC, _NP, 16), jnp.float32),
    mesh=_sc_mesh,
    compiler_params=pltpu.CompilerParams(use_tc_tiling_on_sc=False),
    scratch_types=[
        pltpu.VMEM((_NCHUNK, _CH), jnp.int32),
        pltpu.VMEM((_CH, 16), jnp.float32),
        pltpu.VMEM_SHARED((_NP, 16), jnp.float32),
        pltpu.SemaphoreType.DMA,
    ],
)
def _deg_kernel(dst_hbm, ones_hbm, z16_hbm, cnt_hbm, dst_v, ones_v, acc,
                dsem0):
    c = lax.axis_index("c")
    s = lax.axis_index("s")
    wid = c * _NS + s
    pltpu.sync_copy(dst_hbm.at[wid], dst_v)
    pltpu.sync_copy(ones_hbm, ones_v)
    r0 = s * _RPT
    pltpu.sync_copy(z16_hbm.at[pl.ds(r0, _RPT)], acc.at[pl.ds(r0, _RPT)])
    plsc.subcore_barrier()

    def body(g, carry):
        for k in range(_KC):
            pltpu.async_copy(ones_v, acc.at[dst_v.at[g * _KC + k]],
                             dsem0, add=True)

        @pl.when(g > 0)
        def _():
            for k in range(_KC):
                pltpu.make_async_copy(ones_v, acc.at[dst_v.at[0]],
                                      dsem0).wait()
        return carry

    lax.fori_loop(0, _NG, body, 0)
    for _k in range(_KC):
        pltpu.make_async_copy(ones_v, acc.at[dst_v.at[0]], dsem0).wait()
    plsc.subcore_barrier()
    pltpu.sync_copy(acc.at[pl.ds(r0, _RPT)], cnt_hbm.at[c, pl.ds(r0, _RPT)])


@functools.partial(
    pl.kernel,
    out_type=jax.ShapeDtypeStruct((_NC, _NP, _D), jnp.float32),
    mesh=_sc_mesh,
    scratch_types=[
        pltpu.VMEM((2, _KC, _CH), jnp.int32),
        pltpu.VMEM((2, _KC, _CH), jnp.int32),
        pltpu.VMEM((2, _CH, _D), jnp.float32),
        pltpu.VMEM_SHARED((_NP, _D), jnp.float32),
        pltpu.SemaphoreType.DMA,
        pltpu.SemaphoreType.DMA,
        pltpu.SemaphoreType.DMA,
    ],
)
def _edge_kernel(ys_hbm, src_hbm, dst_hbm, z128_hbm, acc_hbm,
                 sidx, didx, rows_v, acc, gsem0, gsem1, isem):
    c = lax.axis_index("c")
    s = lax.axis_index("s")
    wid = c * _NS + s
    r0 = s * _RPT

    @pl.when(c == 0)
    def _():
        pltpu.sync_copy(ys_hbm.at[pl.ds(r0, _RPT)], acc.at[pl.ds(r0, _RPT)])

    @pl.when(c == 1)
    def _():
        pltpu.sync_copy(z128_hbm.at[pl.ds(r0, _RPT)], acc.at[pl.ds(r0, _RPT)])

    plsc.subcore_barrier()

    gsems = (gsem0, gsem1)
    pltpu.sync_copy(src_hbm.at[wid, pl.ds(0, _KC)], sidx.at[0])
    pltpu.sync_copy(dst_hbm.at[wid, pl.ds(0, _KC)], didx.at[0])
    pltpu.async_copy(ys_hbm.at[sidx.at[0, 0]], rows_v.at[0], gsem0)

    def gbody(g, carry):
        gb = lax.rem(g, 2)
        gn = lax.rem(g + 1, 2)
        gsrc = lax.rem(g + 1, _NG)
        isrc = pltpu.async_copy(src_hbm.at[wid, pl.ds(gsrc * _KC, _KC)],
                                sidx.at[gn], isem)
        idst = pltpu.async_copy(dst_hbm.at[wid, pl.ds(gsrc * _KC, _KC)],
                                didx.at[gn], isem)
        pending = None
        for k in range(_KC):
            cb = k % 2
            nb = (k + 1) % 2
            if k < _KC - 1:
                nxt = pltpu.async_copy(ys_hbm.at[sidx.at[gb, k + 1]],
                                       rows_v.at[nb], gsems[nb])
            else:
                isrc.wait()
                idst.wait()
                nxt = pltpu.async_copy(ys_hbm.at[sidx.at[gn, 0]],
                                       rows_v.at[nb], gsems[nb])
            if pending is None:
                pltpu.make_async_copy(ys_hbm.at[sidx.at[gb, k]],
                                      rows_v.at[cb], gsems[cb]).wait()
            else:
                pending.wait()
            pending = nxt
            pltpu.sync_copy(rows_v.at[cb], acc.at[didx.at[gb, k]], add=True)
        return carry

    lax.fori_loop(0, _NG, gbody, 0)
    pltpu.make_async_copy(ys_hbm.at[sidx.at[0, 0]], rows_v.at[0],
                          gsem0).wait()
    plsc.subcore_barrier()
    pltpu.sync_copy(acc.at[pl.ds(r0, _RPT)], acc_hbm.at[c, pl.ds(r0, _RPT)])


def _dinv_block(cnt_blk):
    deg = cnt_blk[0, :, 0:1] + cnt_blk[1, :, 0:1] + 1.0
    return lax.rsqrt(deg)


def _ka_body(x_ref, wl_ref, bl_ref, w1_ref, cnt_ref, ys_ref):
    h = jnp.maximum(
        jnp.dot(x_ref[...], wl_ref[...], preferred_element_type=jnp.float32)
        + bl_ref[...], 0.0)
    dinv = _dinv_block(cnt_ref[...])
    ys_ref[...] = jnp.dot(h, w1_ref[...],
                          preferred_element_type=jnp.float32) * dinv


def _kb_body(acc_ref, cnt_ref, b_ref, w_ref, ys_ref):
    dinv = _dinv_block(cnt_ref[...])
    h = jnp.maximum(
        (acc_ref[0, :, :] + acc_ref[1, :, :]) * dinv + b_ref[...], 0.0)
    ys_ref[...] = jnp.dot(h, w_ref[...],
                          preferred_element_type=jnp.float32) * dinv


def _kc_body(acc_ref, cnt_ref, b_ref, out_ref):
    dinv = _dinv_block(cnt_ref[...])
    out_ref[...] = jnp.maximum(
        (acc_ref[0, :, :] + acc_ref[1, :, :]) * dinv + b_ref[...], 0.0)


_row_spec = pl.BlockSpec((_B, _D), lambda i: (i, 0))
_mat_spec = pl.BlockSpec((_D, _D), lambda i: (0, 0))
_vec_spec = pl.BlockSpec((_D,), lambda i: (0,))
_cnt_spec = pl.BlockSpec((_NC, _B, 16), lambda i: (0, i, 0))
_acc_spec = pl.BlockSpec((_NC, _B, _D), lambda i: (0, i, 0))
_out_sds = jax.ShapeDtypeStruct((_NP, _D), jnp.float32)


def _ka(x, wl, bl, w1, cnt):
    return pl.pallas_call(
        _ka_body, grid=(_G,),
        in_specs=[_row_spec, _mat_spec, _vec_spec, _mat_spec, _cnt_spec],
        out_specs=_row_spec, out_shape=_out_sds,
    )(x, wl, bl, w1, cnt)


def _kb(acc, cnt, b, w):
    return pl.pallas_call(
        _kb_body, grid=(_G,),
        in_specs=[_acc_spec, _cnt_spec, _vec_spec, _mat_spec],
        out_specs=_row_spec, out_shape=_out_sds,
    )(acc, cnt, b, w)


_BF = 5000


def _kc(acc, cnt, b):
    return pl.pallas_call(
        _kc_body, grid=(_N // _BF,),
        in_specs=[
            pl.BlockSpec((_NC, _BF, _D), lambda i: (0, i, 0)),
            pl.BlockSpec((_NC, _BF, 16), lambda i: (0, i, 0)),
            pl.BlockSpec((_D,), lambda i: (0,)),
        ],
        out_specs=pl.BlockSpec((_BF, _D), lambda i: (i, 0)),
        out_shape=jax.ShapeDtypeStruct((_N, _D), jnp.float32),
    )(acc, cnt, b)


def kernel(features, edge_index, lin_w, lin_b, gcn_w, gcn_b):
    src = edge_index[0].reshape(_NW, _NCHUNK, _CH)
    dst = edge_index[1].reshape(_NW, _NCHUNK, _CH)
    xp = jnp.pad(features, ((0, _NP - _N), (0, 0)))
    ones16 = jnp.ones((_CH, 16), jnp.float32)
    z16 = jnp.zeros((_NP, 16), jnp.float32)
    z128 = jnp.zeros((_NP, _D), jnp.float32)

    cnt = _deg_kernel(dst, ones16, z16)
    ys = _ka(xp, lin_w, lin_b, gcn_w[0], cnt)
    acc = _edge_kernel(ys, src, dst, z128)
    ys = _kb(acc, cnt, gcn_b[0], gcn_w[1])
    acc = _edge_kernel(ys, src, dst, z128)
    return _kc(acc, cnt, gcn_b[1])


# speedup vs baseline: 1.0507x; 1.0048x over previous
import functools

import jax
import jax.numpy as jnp
from jax import lax
from jax.experimental import pallas as pl
from jax.experimental.pallas import tpu as pltpu
from jax.experimental.pallas import tpu_sc as plsc

_N = 10000
_D = 128
_E = 320000
_NC = 2
_NS = 16
_NW = _NC * _NS
_CH = 125
_NCHUNK = 80
_KC = 8
_NG = _NCHUNK // _KC
_SLAB = 632
_LAST = _N - (_NS - 1) * _SLAB

_B = 5000
_G = _N // _B

_sc_mesh = plsc.VectorSubcoreMesh(core_axis_name="c", subcore_axis_name="s")


@functools.partial(
    pl.kernel,
    out_type=jax.ShapeDtypeStruct((_NC, _N, 16), jnp.float32),
    mesh=_sc_mesh,
    compiler_params=pltpu.CompilerParams(use_tc_tiling_on_sc=False),
    scratch_types=[
        pltpu.VMEM((_NCHUNK, _CH), jnp.int32),
        pltpu.VMEM((_CH, 16), jnp.float32),
        pltpu.VMEM_SHARED((_N, 16), jnp.float32),
        pltpu.SemaphoreType.DMA,
    ],
)
def _deg_kernel(dst_hbm, ones_hbm, z16_hbm, cnt_hbm, dst_v, ones_v, acc,
                dsem0):
    c = lax.axis_index("c")
    s = lax.axis_index("s")
    wid = c * _NS + s
    pltpu.sync_copy(dst_hbm.at[wid], dst_v)
    pltpu.sync_copy(ones_hbm, ones_v)
    r0 = s * _SLAB

    @pl.when(s < _NS - 1)
    def _():
        pltpu.sync_copy(z16_hbm.at[pl.ds(r0, _SLAB)],
                        acc.at[pl.ds(r0, _SLAB)])

    @pl.when(s == _NS - 1)
    def _():
        pltpu.sync_copy(z16_hbm.at[pl.ds(r0, _LAST)],
                        acc.at[pl.ds(r0, _LAST)])

    plsc.subcore_barrier()

    def body(g, carry):
        for k in range(_KC):
            pltpu.async_copy(ones_v, acc.at[dst_v.at[g * _KC + k]],
                             dsem0, add=True)

        @pl.when(g > 0)
        def _():
            for k in range(_KC):
                pltpu.make_async_copy(ones_v, acc.at[dst_v.at[0]],
                                      dsem0).wait()
        return carry

    lax.fori_loop(0, _NG, body, 0)
    for _k in range(_KC):
        pltpu.make_async_copy(ones_v, acc.at[dst_v.at[0]], dsem0).wait()
    plsc.subcore_barrier()

    @pl.when(s < _NS - 1)
    def _():
        pltpu.sync_copy(acc.at[pl.ds(r0, _SLAB)],
                        cnt_hbm.at[c, pl.ds(r0, _SLAB)])

    @pl.when(s == _NS - 1)
    def _():
        pltpu.sync_copy(acc.at[pl.ds(r0, _LAST)],
                        cnt_hbm.at[c, pl.ds(r0, _LAST)])


@functools.partial(
    pl.kernel,
    out_type=jax.ShapeDtypeStruct((_NC, _N, _D), jnp.float32),
    mesh=_sc_mesh,
    scratch_types=[
        pltpu.VMEM((2, _KC, _CH), jnp.int32),
        pltpu.VMEM((2, _KC, _CH), jnp.int32),
        pltpu.VMEM((2, _CH, _D), jnp.float32),
        pltpu.VMEM_SHARED((_N, _D), jnp.float32),
        pltpu.SemaphoreType.DMA,
        pltpu.SemaphoreType.DMA,
        pltpu.SemaphoreType.DMA,
    ],
)
def _edge_kernel(ys_hbm, src_hbm, dst_hbm, z128_hbm, acc_hbm,
                 sidx, didx, rows_v, acc, gsem0, gsem1, isem):
    c = lax.axis_index("c")
    s = lax.axis_index("s")
    wid = c * _NS + s
    r0 = s * _SLAB

    @pl.when(s < _NS - 1)
    def _():
        @pl.when(c == 0)
        def _():
            pltpu.sync_copy(ys_hbm.at[pl.ds(r0, _SLAB)],
                            acc.at[pl.ds(r0, _SLAB)])

        @pl.when(c == 1)
        def _():
            pltpu.sync_copy(z128_hbm.at[pl.ds(r0, _SLAB)],
                            acc.at[pl.ds(r0, _SLAB)])

    @pl.when(s == _NS - 1)
    def _():
        @pl.when(c == 0)
        def _():
            pltpu.sync_copy(ys_hbm.at[pl.ds(r0, _LAST)],
                            acc.at[pl.ds(r0, _LAST)])

        @pl.when(c == 1)
        def _():
            pltpu.sync_copy(z128_hbm.at[pl.ds(r0, _LAST)],
                            acc.at[pl.ds(r0, _LAST)])

    plsc.subcore_barrier()

    gsems = (gsem0, gsem1)
    pltpu.sync_copy(src_hbm.at[wid, pl.ds(0, _KC)], sidx.at[0])
    pltpu.sync_copy(dst_hbm.at[wid, pl.ds(0, _KC)], didx.at[0])
    pltpu.async_copy(ys_hbm.at[sidx.at[0, 0]], rows_v.at[0], gsem0)

    def gbody(g, carry):
        gb = lax.rem(g, 2)
        gn = lax.rem(g + 1, 2)
        gsrc = lax.rem(g + 1, _NG)
        isrc = pltpu.async_copy(src_hbm.at[wid, pl.ds(gsrc * _KC, _KC)],
                                sidx.at[gn], isem)
        idst = pltpu.async_copy(dst_hbm.at[wid, pl.ds(gsrc * _KC, _KC)],
                                didx.at[gn], isem)
        pending = None
        for k in range(_KC):
            cb = k % 2
            nb = (k + 1) % 2
            if k < _KC - 1:
                nxt = pltpu.async_copy(ys_hbm.at[sidx.at[gb, k + 1]],
                                       rows_v.at[nb], gsems[nb])
            else:
                isrc.wait()
                idst.wait()
                nxt = pltpu.async_copy(ys_hbm.at[sidx.at[gn, 0]],
                                       rows_v.at[nb], gsems[nb])
            if pending is None:
                pltpu.make_async_copy(ys_hbm.at[sidx.at[gb, k]],
                                      rows_v.at[cb], gsems[cb]).wait()
            else:
                pending.wait()
            pending = nxt
            pltpu.sync_copy(rows_v.at[cb], acc.at[didx.at[gb, k]], add=True)
        return carry

    lax.fori_loop(0, _NG, gbody, 0)
    pltpu.make_async_copy(ys_hbm.at[sidx.at[0, 0]], rows_v.at[0],
                          gsem0).wait()
    plsc.subcore_barrier()

    @pl.when(s < _NS - 1)
    def _():
        pltpu.sync_copy(acc.at[pl.ds(r0, _SLAB)],
                        acc_hbm.at[c, pl.ds(r0, _SLAB)])

    @pl.when(s == _NS - 1)
    def _():
        pltpu.sync_copy(acc.at[pl.ds(r0, _LAST)],
                        acc_hbm.at[c, pl.ds(r0, _LAST)])


def _dinv_block(cnt_blk):
    deg = cnt_blk[0, :, 0:1] + cnt_blk[1, :, 0:1] + 1.0
    return lax.rsqrt(deg)


def _ka_body(x_ref, wl_ref, bl_ref, w1_ref, cnt_ref, ys_ref):
    h = jnp.maximum(
        jnp.dot(x_ref[...], wl_ref[...], preferred_element_type=jnp.float32)
        + bl_ref[...], 0.0)
    dinv = _dinv_block(cnt_ref[...])
    ys_ref[...] = jnp.dot(h, w1_ref[...],
                          preferred_element_type=jnp.float32) * dinv


def _kb_body(acc_ref, cnt_ref, b_ref, w_ref, ys_ref):
    dinv = _dinv_block(cnt_ref[...])
    h = jnp.maximum(
        (acc_ref[0, :, :] + acc_ref[1, :, :]) * dinv + b_ref[...], 0.0)
    ys_ref[...] = jnp.dot(h, w_ref[...],
                          preferred_element_type=jnp.float32) * dinv


def _kc_body(acc_ref, cnt_ref, b_ref, out_ref):
    dinv = _dinv_block(cnt_ref[...])
    out_ref[...] = jnp.maximum(
        (acc_ref[0, :, :] + acc_ref[1, :, :]) * dinv + b_ref[...], 0.0)


_row_spec = pl.BlockSpec((_B, _D), lambda i: (i, 0))
_mat_spec = pl.BlockSpec((_D, _D), lambda i: (0, 0))
_vec_spec = pl.BlockSpec((_D,), lambda i: (0,))
_cnt_spec = pl.BlockSpec((_NC, _B, 16), lambda i: (0, i, 0))
_acc_spec = pl.BlockSpec((_NC, _B, _D), lambda i: (0, i, 0))
_out_sds = jax.ShapeDtypeStruct((_N, _D), jnp.float32)


def _ka(x, wl, bl, w1, cnt):
    return pl.pallas_call(
        _ka_body, grid=(_G,),
        in_specs=[_row_spec, _mat_spec, _vec_spec, _mat_spec, _cnt_spec],
        out_specs=_row_spec, out_shape=_out_sds,
    )(x, wl, bl, w1, cnt)


def _kb(acc, cnt, b, w):
    return pl.pallas_call(
        _kb_body, grid=(_G,),
        in_specs=[_acc_spec, _cnt_spec, _vec_spec, _mat_spec],
        out_specs=_row_spec, out_shape=_out_sds,
    )(acc, cnt, b, w)


_BF = 5000


def _kc(acc, cnt, b):
    return pl.pallas_call(
        _kc_body, grid=(_N // _BF,),
        in_specs=[
            pl.BlockSpec((_NC, _BF, _D), lambda i: (0, i, 0)),
            pl.BlockSpec((_NC, _BF, 16), lambda i: (0, i, 0)),
            pl.BlockSpec((_D,), lambda i: (0,)),
        ],
        out_specs=pl.BlockSpec((_BF, _D), lambda i: (i, 0)),
        out_shape=jax.ShapeDtypeStruct((_N, _D), jnp.float32),
    )(acc, cnt, b)


def kernel(features, edge_index, lin_w, lin_b, gcn_w, gcn_b):
    src = edge_index[0].reshape(_NW, _NCHUNK, _CH)
    dst = edge_index[1].reshape(_NW, _NCHUNK, _CH)
    ones16 = jnp.ones((_CH, 16), jnp.float32)
    z16 = jnp.zeros((_N, 16), jnp.float32)
    z128 = jnp.zeros((_N, _D), jnp.float32)

    cnt = _deg_kernel(dst, ones16, z16)
    ys = _ka(features, lin_w, lin_b, gcn_w[0], cnt)
    acc = _edge_kernel(ys, src, dst, z128)
    ys = _kb(acc, cnt, gcn_b[0], gcn_w[1])
    acc = _edge_kernel(ys, src, dst, z128)
    return _kc(acc, cnt, gcn_b[1])


# speedup vs baseline: 1.0737x; 1.0219x over previous
import functools

import jax
import jax.numpy as jnp
from jax import lax
from jax.experimental import pallas as pl
from jax.experimental.pallas import tpu as pltpu
from jax.experimental.pallas import tpu_sc as plsc

_N = 10000
_D = 128
_E = 320000
_NC = 2
_NS = 16
_NW = _NC * _NS
_CH = 125
_NCHUNK = 80
_KC = 8
_NG = _NCHUNK // _KC
_SLAB = 632
_LAST = _N - (_NS - 1) * _SLAB

_B = 5000
_G = _N // _B

_sc_mesh = plsc.VectorSubcoreMesh(core_axis_name="c", subcore_axis_name="s")


@functools.partial(
    pl.kernel,
    out_type=jax.ShapeDtypeStruct((_NC, _N, 16), jnp.float32),
    mesh=_sc_mesh,
    compiler_params=pltpu.CompilerParams(use_tc_tiling_on_sc=False),
    scratch_types=[
        pltpu.VMEM((_NCHUNK, _CH), jnp.int32),
        pltpu.VMEM((_CH, 16), jnp.float32),
        pltpu.VMEM_SHARED((_N, 16), jnp.float32),
        pltpu.SemaphoreType.DMA,
    ],
)
def _deg_kernel(dst_hbm, ones_hbm, z16_hbm, cnt_hbm, dst_v, ones_v, acc,
                dsem0):
    c = lax.axis_index("c")
    s = lax.axis_index("s")
    wid = c * _NS + s
    pltpu.sync_copy(dst_hbm.at[wid], dst_v)
    pltpu.sync_copy(ones_hbm, ones_v)
    r0 = s * _SLAB

    @pl.when(s < _NS - 1)
    def _():
        pltpu.sync_copy(z16_hbm.at[pl.ds(r0, _SLAB)],
                        acc.at[pl.ds(r0, _SLAB)])

    @pl.when(s == _NS - 1)
    def _():
        pltpu.sync_copy(z16_hbm.at[pl.ds(r0, _LAST)],
                        acc.at[pl.ds(r0, _LAST)])

    plsc.subcore_barrier()

    def body(g, carry):
        for k in range(_KC):
            pltpu.async_copy(ones_v, acc.at[dst_v.at[g * _KC + k]],
                             dsem0, add=True)

        @pl.when(g > 0)
        def _():
            for k in range(_KC):
                pltpu.make_async_copy(ones_v, acc.at[dst_v.at[0]],
                                      dsem0).wait()
        return carry

    lax.fori_loop(0, _NG, body, 0)
    for _k in range(_KC):
        pltpu.make_async_copy(ones_v, acc.at[dst_v.at[0]], dsem0).wait()
    plsc.subcore_barrier()

    @pl.when(s < _NS - 1)
    def _():
        pltpu.sync_copy(acc.at[pl.ds(r0, _SLAB)],
                        cnt_hbm.at[c, pl.ds(r0, _SLAB)])

    @pl.when(s == _NS - 1)
    def _():
        pltpu.sync_copy(acc.at[pl.ds(r0, _LAST)],
                        cnt_hbm.at[c, pl.ds(r0, _LAST)])


@functools.partial(
    pl.kernel,
    out_type=jax.ShapeDtypeStruct((_NC, _N, _D), jnp.float32),
    mesh=_sc_mesh,
    scratch_types=[
        pltpu.VMEM((2, _KC, _CH), jnp.int32),
        pltpu.VMEM((2, _KC, _CH), jnp.int32),
        pltpu.VMEM((2, _CH, _D), jnp.float32),
        pltpu.VMEM_SHARED((_N, _D), jnp.float32),
        pltpu.SemaphoreType.DMA,
        pltpu.SemaphoreType.DMA,
        pltpu.SemaphoreType.DMA,
    ],
)
def _edge_kernel(ys_hbm, src_hbm, dst_hbm, z128_hbm, acc_hbm,
                 sidx, didx, rows_v, acc, gsem0, gsem1, isem):
    c = lax.axis_index("c")
    s = lax.axis_index("s")
    wid = c * _NS + s
    r0 = s * _SLAB

    @pl.when(s < _NS - 1)
    def _():
        @pl.when(c == 0)
        def _():
            pltpu.async_copy(ys_hbm.at[pl.ds(r0, _SLAB)],
                             acc.at[pl.ds(r0, _SLAB)], isem)

        @pl.when(c == 1)
        def _():
            pltpu.async_copy(z128_hbm.at[pl.ds(r0, _SLAB)],
                             acc.at[pl.ds(r0, _SLAB)], isem)

    @pl.when(s == _NS - 1)
    def _():
        @pl.when(c == 0)
        def _():
            pltpu.async_copy(ys_hbm.at[pl.ds(r0, _LAST)],
                             acc.at[pl.ds(r0, _LAST)], isem)

        @pl.when(c == 1)
        def _():
            pltpu.async_copy(z128_hbm.at[pl.ds(r0, _LAST)],
                             acc.at[pl.ds(r0, _LAST)], isem)

    gsems = (gsem0, gsem1)
    pltpu.sync_copy(src_hbm.at[wid, pl.ds(0, _KC)], sidx.at[0])
    pltpu.sync_copy(dst_hbm.at[wid, pl.ds(0, _KC)], didx.at[0])
    pltpu.async_copy(ys_hbm.at[sidx.at[0, 0]], rows_v.at[0], gsem0)

    @pl.when(s < _NS - 1)
    def _():
        pltpu.make_async_copy(ys_hbm.at[pl.ds(r0, _SLAB)],
                              acc.at[pl.ds(r0, _SLAB)], isem).wait()

    @pl.when(s == _NS - 1)
    def _():
        pltpu.make_async_copy(ys_hbm.at[pl.ds(r0, _LAST)],
                              acc.at[pl.ds(r0, _LAST)], isem).wait()

    plsc.subcore_barrier()

    def gbody(g, carry):
        gb = lax.rem(g, 2)
        gn = lax.rem(g + 1, 2)
        gsrc = lax.rem(g + 1, _NG)
        isrc = pltpu.async_copy(src_hbm.at[wid, pl.ds(gsrc * _KC, _KC)],
                                sidx.at[gn], isem)
        idst = pltpu.async_copy(dst_hbm.at[wid, pl.ds(gsrc * _KC, _KC)],
                                didx.at[gn], isem)
        pending = None
        for k in range(_KC):
            cb = k % 2
            nb = (k + 1) % 2
            if k < _KC - 1:
                nxt = pltpu.async_copy(ys_hbm.at[sidx.at[gb, k + 1]],
                                       rows_v.at[nb], gsems[nb])
            else:
                isrc.wait()
                idst.wait()
                nxt = pltpu.async_copy(ys_hbm.at[sidx.at[gn, 0]],
                                       rows_v.at[nb], gsems[nb])
            if pending is None:
                pltpu.make_async_copy(ys_hbm.at[sidx.at[gb, k]],
                                      rows_v.at[cb], gsems[cb]).wait()
            else:
                pending.wait()
            pending = nxt
            pltpu.sync_copy(rows_v.at[cb], acc.at[didx.at[gb, k]], add=True)
        return carry

    lax.fori_loop(0, _NG, gbody, 0)
    pltpu.make_async_copy(ys_hbm.at[sidx.at[0, 0]], rows_v.at[0],
                          gsem0).wait()
    plsc.subcore_barrier()

    @pl.when(s < _NS - 1)
    def _():
        pltpu.sync_copy(acc.at[pl.ds(r0, _SLAB)],
                        acc_hbm.at[c, pl.ds(r0, _SLAB)])

    @pl.when(s == _NS - 1)
    def _():
        pltpu.sync_copy(acc.at[pl.ds(r0, _LAST)],
                        acc_hbm.at[c, pl.ds(r0, _LAST)])


def _dinv_block(cnt_blk):
    deg = cnt_blk[0, :, 0:1] + cnt_blk[1, :, 0:1] + 1.0
    return lax.rsqrt(deg)


def _ka_body(x_ref, wl_ref, bl_ref, w1_ref, cnt_ref, ys_ref):
    h = jnp.maximum(
        jnp.dot(x_ref[...], wl_ref[...], preferred_element_type=jnp.float32)
        + bl_ref[...], 0.0)
    dinv = _dinv_block(cnt_ref[...])
    ys_ref[...] = jnp.dot(h, w1_ref[...],
                          preferred_element_type=jnp.float32) * dinv


def _kb_body(acc_ref, cnt_ref, b_ref, w_ref, ys_ref):
    dinv = _dinv_block(cnt_ref[...])
    h = jnp.maximum(
        (acc_ref[0, :, :] + acc_ref[1, :, :]) * dinv + b_ref[...], 0.0)
    ys_ref[...] = jnp.dot(h, w_ref[...],
                          preferred_element_type=jnp.float32) * dinv


def _kc_body(acc_ref, cnt_ref, b_ref, out_ref):
    dinv = _dinv_block(cnt_ref[...])
    out_ref[...] = jnp.maximum(
        (acc_ref[0, :, :] + acc_ref[1, :, :]) * dinv + b_ref[...], 0.0)


_row_spec = pl.BlockSpec((_B, _D), lambda i: (i, 0))
_mat_spec = pl.BlockSpec((_D, _D), lambda i: (0, 0))
_vec_spec = pl.BlockSpec((_D,), lambda i: (0,))
_cnt_spec = pl.BlockSpec((_NC, _B, 16), lambda i: (0, i, 0))
_acc_spec = pl.BlockSpec((_NC, _B, _D), lambda i: (0, i, 0))
_out_sds = jax.ShapeDtypeStruct((_N, _D), jnp.float32)


def _ka(x, wl, bl, w1, cnt):
    return pl.pallas_call(
        _ka_body, grid=(_G,),
        in_specs=[_row_spec, _mat_spec, _vec_spec, _mat_spec, _cnt_spec],
        out_specs=_row_spec, out_shape=_out_sds,
    )(x, wl, bl, w1, cnt)


def _kb(acc, cnt, b, w):
    return pl.pallas_call(
        _kb_body, grid=(_G,),
        in_specs=[_acc_spec, _cnt_spec, _vec_spec, _mat_spec],
        out_specs=_row_spec, out_shape=_out_sds,
    )(acc, cnt, b, w)


_BF = 5000


def _kc(acc, cnt, b):
    return pl.pallas_call(
        _kc_body, grid=(_N // _BF,),
        in_specs=[
            pl.BlockSpec((_NC, _BF, _D), lambda i: (0, i, 0)),
            pl.BlockSpec((_NC, _BF, 16), lambda i: (0, i, 0)),
            pl.BlockSpec((_D,), lambda i: (0,)),
        ],
        out_specs=pl.BlockSpec((_BF, _D), lambda i: (i, 0)),
        out_shape=jax.ShapeDtypeStruct((_N, _D), jnp.float32),
    )(acc, cnt, b)


def kernel(features, edge_index, lin_w, lin_b, gcn_w, gcn_b):
    src = edge_index[0].reshape(_NW, _NCHUNK, _CH)
    dst = edge_index[1].reshape(_NW, _NCHUNK, _CH)
    ones16 = jnp.ones((_CH, 16), jnp.float32)
    z16 = jnp.zeros((_N, 16), jnp.float32)
    z128 = jnp.zeros((_N, _D), jnp.float32)

    cnt = _deg_kernel(dst, ones16, z16)
    ys = _ka(features, lin_w, lin_b, gcn_w[0], cnt)
    acc = _edge_kernel(ys, src, dst, z128)
    ys = _kb(acc, cnt, gcn_b[0], gcn_w[1])
    acc = _edge_kernel(ys, src, dst, z128)
    return _kc(acc, cnt, gcn_b[1])
